# baseline (speedup 1.0000x reference)
import functools

import jax
import jax.numpy as jnp
from jax import lax
from jax.experimental import pallas as pl
from jax.experimental.pallas import tpu as pltpu
from jax.experimental.pallas import tpu_sc as plsc

_NC = 2
_NS = 16
_NW = _NC * _NS

_L = 16
_TB = 128

_CHUNK = 104
_SUPER = 4
_GROUP = _CHUNK * _SUPER


def _tr_body(n_rows, dim, wt_hbm, tail_hbm, wp_hbm, in_v, out_v):
    wid = lax.axis_index("s") * _NC + lax.axis_index("c")
    n_full = n_rows // _TB
    rem = n_rows - n_full * _TB

    iotas = [lax.iota(jnp.int32, _L) + q * _L for q in range(dim // _L)]

    def shuffle(n_lines):
        for p in range(n_lines):
            for h in range(2):
                col = jnp.full((_L,), 2 * p + h, jnp.int32)
                for q in range(dim // _L):
                    out_v[p, pl.ds(h * dim + q * _L, _L)] = plsc.load_gather(
                        in_v, [iotas[q], col]
                    )

    def block_body(i, _):
        tb = i * _NW + wid
        r0 = tb * _TB
        pltpu.sync_copy(wt_hbm.at[:, pl.ds(r0, _TB)], in_v)
        shuffle(_TB // 2)
        pltpu.sync_copy(out_v, wp_hbm.at[pl.ds(tb * (_TB // 2), _TB // 2)])
        return 0

    n_iter = (n_full - wid + _NW - 1) // _NW
    lax.fori_loop(0, n_iter, block_body, 0)

    if rem:
        @pl.when(wid == 0)
        def _():
            pltpu.sync_copy(tail_hbm, in_v)
            shuffle(rem // 2)
            pltpu.sync_copy(
                out_v.at[pl.ds(0, rem // 2)],
                wp_hbm.at[pl.ds(n_full * (_TB // 2), rem // 2)],
            )


def _emb_body(n_groups, fields, dim, table_hbm, idx_hbm, out_hbm,
              idx_v, buf0, buf1, gs0, gs1, os0, os1):
    wid = lax.axis_index("s") * _NC + lax.axis_index("c")
    per_w = n_groups * _GROUP
    b_per_g = _GROUP // fields
    base_b = wid * (per_w // fields)
    pltpu.sync_copy(idx_hbm.at[wid], idx_v)

    bufs = (buf0, buf1)
    gsems = (gs0, gs1)
    osems = (os0, os1)

    def fire(g):
        b = g % 2
        return [
            pltpu.async_copy(
                table_hbm.at[idx_v.at[g * _SUPER + k]],
                bufs[b].at[pl.ds(k * _CHUNK, _CHUNK)],
                gsems[b],
            )
            for k in range(_SUPER)
        ]

    def drain_out(g):
        b = g % 2
        return [
            pltpu.async_copy(
                bufs[b].at[pl.ds(j * fields, fields)],
                out_hbm.at[base_b + g * b_per_g + j],
                osems[b],
            )
            for j in range(b_per_g)
        ]

    pend = {0: fire(0), 1: fire(1)}
    tail = []
    for g in range(n_groups):
        b = g % 2
        for cp in pend[b]:
            cp.wait()
        ocps = drain_out(g)
        if g + 2 < n_groups:
            for ocp in ocps:
                ocp.wait()
            pend[b] = fire(g + 2)
        else:
            tail.extend(ocps)
    for ocp in tail:
        ocp.wait()


@functools.partial(jax.jit, static_argnums=(2, 3, 4))
def _emb_lookup(weights, flat_idx, n_groups, fields, dim):
    total = flat_idx.shape[0]
    batch = total // fields
    n_rows = weights.shape[0]
    idx3d = flat_idx.reshape(_NW, (total // _NW) // _CHUNK, _CHUNK)
    mesh = plsc.VectorSubcoreMesh(core_axis_name="c", subcore_axis_name="s")

    transpose = pl.kernel(
        functools.partial(_tr_body, n_rows, dim),
        out_type=jax.ShapeDtypeStruct((n_rows // 2, 2 * dim), jnp.float32),
        mesh=mesh,
        scratch_types=[
            pltpu.VMEM((dim, _TB), jnp.float32),
            pltpu.VMEM((_TB // 2, 2 * dim), jnp.float32),
        ],
        compiler_params=pltpu.CompilerParams(
            use_tc_tiling_on_sc=True, needs_layout_passes=False
        ),
    )
    wt = weights.T
    n_full = n_rows // _TB
    rem = n_rows - n_full * _TB
    tail = lax.slice(wt, (0, n_full * _TB), (dim, n_rows))
    tail = jnp.pad(tail, ((0, 0), (0, _TB - rem)))
    wrow = transpose(wt, tail).reshape(n_rows, dim)

    lookup = pl.kernel(
        functools.partial(_emb_body, n_groups, fields, dim),
        out_type=jax.ShapeDtypeStruct((batch, fields, dim), jnp.float32),
        mesh=mesh,
        scratch_types=[
            pltpu.VMEM(((total // _NW) // _CHUNK, _CHUNK), jnp.int32),
            pltpu.VMEM((_GROUP, dim), jnp.float32),
            pltpu.VMEM((_GROUP, dim), jnp.float32),
            pltpu.SemaphoreType.DMA,
            pltpu.SemaphoreType.DMA,
            pltpu.SemaphoreType.DMA,
            pltpu.SemaphoreType.DMA,
        ],
        compiler_params=pltpu.CompilerParams(use_tc_tiling_on_sc=False),
    )
    return lookup(wrow, idx3d)


def kernel(weights, token_ids):
    batch, fields = token_ids.shape
    dim = weights.shape[1]
    total = batch * fields
    per_w = total // _NW
    n_groups = per_w // _GROUP
    flat = token_ids.reshape(total)
    return _emb_lookup(weights, flat, n_groups, fields, dim)


# speedup vs baseline: 1.4397x; 1.4397x over previous
import functools

import jax
import jax.numpy as jnp
from jax import lax
from jax.experimental import pallas as pl
from jax.experimental.pallas import tpu as pltpu
from jax.experimental.pallas import tpu_sc as plsc

_NC = 2
_NS = 16
_NW = _NC * _NS

_L = 16
_TB = 128

_CHUNK = 104
_SUPER = 4
_GROUP = _CHUNK * _SUPER


def _tr_body(n_rows, dim, wt_hbm, tail_hbm, wp_hbm,
             in0, in1, ou0, ou1, is0, is1, os0, os1):
    wid = lax.axis_index("s") * _NC + lax.axis_index("c")
    n_full = n_rows // _TB
    rem = n_rows - n_full * _TB
    n_iter = (n_full + _NW - 1) // _NW
    if n_iter % 2:
        n_iter += 1

    ins = (in0, in1)
    outs = (ou0, ou1)
    isems = (is0, is1)
    osems = (os0, os1)

    iota = lax.iota(jnp.int32, _L)
    rows_g = [(iota + g * _L) // 2 for g in range(_TB // _L)]
    cols_g = [((iota + g * _L) % 2) * dim for g in range(_TB // _L)]

    def start_in(i, b):
        tb = i * _NW + wid
        @pl.when(tb < n_full)
        def _():
            pltpu.make_async_copy(
                wt_hbm.at[:, pl.ds(tb * _TB, _TB)], ins[b], isems[b]
            ).start()

    def shuffle(src, dst):
        def c_body(c, _):
            for g in range(_TB // _L):
                v = src[c, pl.ds(g * _L, _L)]
                plsc.store_scatter(dst, [rows_g[g], cols_g[g] + c], v)
            return 0
        lax.fori_loop(0, dim, c_body, 0)

    for b in range(2):
        start_in(b, b)

    def pair_body(it, _):
        for b in range(2):
            i = it * 2 + b
            tb = i * _NW + wid

            @pl.when(tb < n_full)
            def _():
                pltpu.make_async_copy(
                    wt_hbm.at[:, pl.ds(tb * _TB, _TB)], ins[b], isems[b]
                ).wait()

            @pl.when(jnp.logical_and(tb < n_full, i >= 2))
            def _():
                pltpu.make_async_copy(
                    outs[b],
                    wp_hbm.at[pl.ds((tb - 2 * _NW) * (_TB // 2), _TB // 2)],
                    osems[b],
                ).wait()

            @pl.when(tb < n_full)
            def _():
                shuffle(ins[b], outs[b])
                pltpu.make_async_copy(
                    outs[b], wp_hbm.at[pl.ds(tb * (_TB // 2), _TB // 2)],
                    osems[b],
                ).start()
                start_in(i + 2, b)
        return 0

    lax.fori_loop(0, n_iter // 2, pair_body, 0)

    for b in range(2):
        @pl.when(b * _NW + wid < n_full)
        def _():
            pltpu.make_async_copy(
                outs[b], wp_hbm.at[pl.ds(0, _TB // 2)], osems[b]
            ).wait()

    if rem:
        @pl.when(wid == 0)
        def _():
            pltpu.sync_copy(tail_hbm, in0)
            def c_body(c, _):
                for g in range(rem // _L):
                    v = in0[c, pl.ds(g * _L, _L)]
                    plsc.store_scatter(ou0, [rows_g[g], cols_g[g] + c], v)
                return 0
            lax.fori_loop(0, dim, c_body, 0)
            pltpu.sync_copy(
                ou0.at[pl.ds(0, rem // 2)],
                wp_hbm.at[pl.ds(n_full * (_TB // 2), rem // 2)],
            )


def _emb_body(n_groups, fields, dim, table_hbm, idx_hbm, out_hbm,
              idx_v, buf0, buf1, gs0, gs1, os0, os1):
    wid = lax.axis_index("s") * _NC + lax.axis_index("c")
    per_w = n_groups * _GROUP
    b_per_g = _GROUP // fields
    base_b = wid * (per_w // fields)
    pltpu.sync_copy(idx_hbm.at[wid], idx_v)

    bufs = (buf0, buf1)
    gsems = (gs0, gs1)
    osems = (os0, os1)

    def fire(g):
        b = g % 2
        return [
            pltpu.async_copy(
                table_hbm.at[idx_v.at[g * _SUPER + k]],
                bufs[b].at[pl.ds(k * _CHUNK, _CHUNK)],
                gsems[b],
            )
            for k in range(_SUPER)
        ]

    def drain_out(g):
        b = g % 2
        return [
            pltpu.async_copy(
                bufs[b].at[pl.ds(j * fields, fields)],
                out_hbm.at[base_b + g * b_per_g + j],
                osems[b],
            )
            for j in range(b_per_g)
        ]

    pend = {0: fire(0), 1: fire(1)}
    tail = []
    for g in range(n_groups):
        b = g % 2
        for cp in pend[b]:
            cp.wait()
        ocps = drain_out(g)
        if g + 2 < n_groups:
            for ocp in ocps:
                ocp.wait()
            pend[b] = fire(g + 2)
        else:
            tail.extend(ocps)
    for ocp in tail:
        ocp.wait()


@functools.partial(jax.jit, static_argnums=(2, 3, 4))
def _emb_lookup(weights, flat_idx, n_groups, fields, dim):
    total = flat_idx.shape[0]
    batch = total // fields
    n_rows = weights.shape[0]
    idx3d = flat_idx.reshape(_NW, (total // _NW) // _CHUNK, _CHUNK)
    mesh = plsc.VectorSubcoreMesh(core_axis_name="c", subcore_axis_name="s")

    transpose = pl.kernel(
        functools.partial(_tr_body, n_rows, dim),
        out_type=jax.ShapeDtypeStruct((n_rows // 2, 2 * dim), jnp.float32),
        mesh=mesh,
        scratch_types=[
            pltpu.VMEM((dim, _TB), jnp.float32),
            pltpu.VMEM((dim, _TB), jnp.float32),
            pltpu.VMEM((_TB // 2, 2 * dim), jnp.float32),
            pltpu.VMEM((_TB // 2, 2 * dim), jnp.float32),
            pltpu.SemaphoreType.DMA,
            pltpu.SemaphoreType.DMA,
            pltpu.SemaphoreType.DMA,
            pltpu.SemaphoreType.DMA,
        ],
        compiler_params=pltpu.CompilerParams(
            use_tc_tiling_on_sc=True, needs_layout_passes=False
        ),
    )
    wt = weights.T
    n_full = n_rows // _TB
    rem = n_rows - n_full * _TB
    tail = lax.slice(wt, (0, n_full * _TB), (dim, n_rows))
    tail = jnp.pad(tail, ((0, 0), (0, _TB - rem)))
    wrow = transpose(wt, tail).reshape(n_rows, dim)

    lookup = pl.kernel(
        functools.partial(_emb_body, n_groups, fields, dim),
        out_type=jax.ShapeDtypeStruct((batch, fields, dim), jnp.float32),
        mesh=mesh,
        scratch_types=[
            pltpu.VMEM(((total // _NW) // _CHUNK, _CHUNK), jnp.int32),
            pltpu.VMEM((_GROUP, dim), jnp.float32),
            pltpu.VMEM((_GROUP, dim), jnp.float32),
            pltpu.SemaphoreType.DMA,
            pltpu.SemaphoreType.DMA,
            pltpu.SemaphoreType.DMA,
            pltpu.SemaphoreType.DMA,
        ],
        compiler_params=pltpu.CompilerParams(use_tc_tiling_on_sc=False),
    )
    return lookup(wrow, idx3d)


def kernel(weights, token_ids):
    batch, fields = token_ids.shape
    dim = weights.shape[1]
    total = batch * fields
    per_w = total // _NW
    n_groups = per_w // _GROUP
    flat = token_ids.reshape(total)
    return _emb_lookup(weights, flat, n_groups, fields, dim)


# speedup vs baseline: 1.5890x; 1.1037x over previous
import functools

import jax
import jax.numpy as jnp
from jax import lax
from jax.experimental import pallas as pl
from jax.experimental.pallas import tpu as pltpu
from jax.experimental.pallas import tpu_sc as plsc

_NC = 2
_NS = 16
_NW = _NC * _NS

_L = 16
_TB = 128

_CHUNK = 104
_SUPER = 4
_GROUP = _CHUNK * _SUPER


def _tr_body(n_rows, dim, wt_hbm, tail_hbm, wp_hbm,
             in0, in1, ou0, ou1, is0, is1, os0, os1):
    wid = lax.axis_index("s") * _NC + lax.axis_index("c")
    n_full = n_rows // _TB
    rem = n_rows - n_full * _TB
    n_iter = (n_full + _NW - 1) // _NW
    if n_iter % 2:
        n_iter += 1

    ins = (in0, in1)
    outs = (ou0, ou1)
    isems = (is0, is1)
    osems = (os0, os1)

    iota = lax.iota(jnp.int32, _L)
    rows_q = [iota + q * _L for q in range(dim // _L)]

    def start_in(i, b):
        tb = i * _NW + wid
        @pl.when(tb < n_full)
        def _():
            pltpu.make_async_copy(
                wt_hbm.at[:, pl.ds(tb * _TB, _TB)], ins[b], isems[b]
            ).start()

    nq = dim // _L

    def shuffle(src, dst):
        def p_body(p, _):
            vals = []
            for h in range(2):
                col = jnp.full((_L,), 2 * p + h, jnp.int32)
                for q in range(nq):
                    vals.append(plsc.load_gather(src, [rows_q[q], col]))
            for h in range(2):
                for q in range(nq):
                    dst[p, pl.ds(h * dim + q * _L, _L)] = vals[h * nq + q]
            return 0

        lax.fori_loop(0, _TB // 2, p_body, 0)

    for b in range(2):
        start_in(b, b)

    def pair_body(it, _):
        for b in range(2):
            i = it * 2 + b
            tb = i * _NW + wid

            @pl.when(tb < n_full)
            def _():
                pltpu.make_async_copy(
                    wt_hbm.at[:, pl.ds(tb * _TB, _TB)], ins[b], isems[b]
                ).wait()

            @pl.when(jnp.logical_and(tb < n_full, i >= 2))
            def _():
                pltpu.make_async_copy(
                    outs[b],
                    wp_hbm.at[pl.ds((tb - 2 * _NW) * (_TB // 2), _TB // 2)],
                    osems[b],
                ).wait()

            @pl.when(tb < n_full)
            def _():
                shuffle(ins[b], outs[b])
                pltpu.make_async_copy(
                    outs[b], wp_hbm.at[pl.ds(tb * (_TB // 2), _TB // 2)],
                    osems[b],
                ).start()
                start_in(i + 2, b)
        return 0

    lax.fori_loop(0, n_iter // 2, pair_body, 0)

    for b in range(2):
        @pl.when(b * _NW + wid < n_full)
        def _():
            pltpu.make_async_copy(
                outs[b], wp_hbm.at[pl.ds(0, _TB // 2)], osems[b]
            ).wait()

    if rem:
        @pl.when(wid == 0)
        def _():
            pltpu.sync_copy(tail_hbm, in0)
            def p_body(p, _):
                for h in range(2):
                    col = jnp.full((_L,), 2 * p + h, jnp.int32)
                    for q in range(dim // _L):
                        ou0[p, pl.ds(h * dim + q * _L, _L)] = plsc.load_gather(
                            in0, [rows_q[q], col]
                        )
                return 0
            lax.fori_loop(0, rem // 2, p_body, 0)
            pltpu.sync_copy(
                ou0.at[pl.ds(0, rem // 2)],
                wp_hbm.at[pl.ds(n_full * (_TB // 2), rem // 2)],
            )


def _emb_body(n_groups, fields, dim, table_hbm, idx_hbm, out_hbm,
              idx_v, buf0, buf1, gs0, gs1, os0, os1):
    wid = lax.axis_index("s") * _NC + lax.axis_index("c")
    per_w = n_groups * _GROUP
    b_per_g = _GROUP // fields
    base_b = wid * (per_w // fields)
    pltpu.sync_copy(idx_hbm.at[wid], idx_v)

    bufs = (buf0, buf1)
    gsems = (gs0, gs1)
    osems = (os0, os1)

    def fire(g):
        b = g % 2
        return [
            pltpu.async_copy(
                table_hbm.at[idx_v.at[g * _SUPER + k]],
                bufs[b].at[pl.ds(k * _CHUNK, _CHUNK)],
                gsems[b],
            )
            for k in range(_SUPER)
        ]

    def drain_out(g):
        b = g % 2
        return [
            pltpu.async_copy(
                bufs[b].at[pl.ds(j * fields, fields)],
                out_hbm.at[base_b + g * b_per_g + j],
                osems[b],
            )
            for j in range(b_per_g)
        ]

    pend = {0: fire(0), 1: fire(1)}
    tail = []
    for g in range(n_groups):
        b = g % 2
        for cp in pend[b]:
            cp.wait()
        ocps = drain_out(g)
        if g + 2 < n_groups:
            for ocp in ocps:
                ocp.wait()
            pend[b] = fire(g + 2)
        else:
            tail.extend(ocps)
    for ocp in tail:
        ocp.wait()


@functools.partial(jax.jit, static_argnums=(2, 3, 4))
def _emb_lookup(weights, flat_idx, n_groups, fields, dim):
    total = flat_idx.shape[0]
    batch = total // fields
    n_rows = weights.shape[0]
    idx3d = flat_idx.reshape(_NW, (total // _NW) // _CHUNK, _CHUNK)
    mesh = plsc.VectorSubcoreMesh(core_axis_name="c", subcore_axis_name="s")

    transpose = pl.kernel(
        functools.partial(_tr_body, n_rows, dim),
        out_type=jax.ShapeDtypeStruct((n_rows // 2, 2 * dim), jnp.float32),
        mesh=mesh,
        scratch_types=[
            pltpu.VMEM((dim, _TB), jnp.float32),
            pltpu.VMEM((dim, _TB), jnp.float32),
            pltpu.VMEM((_TB // 2, 2 * dim), jnp.float32),
            pltpu.VMEM((_TB // 2, 2 * dim), jnp.float32),
            pltpu.SemaphoreType.DMA,
            pltpu.SemaphoreType.DMA,
            pltpu.SemaphoreType.DMA,
            pltpu.SemaphoreType.DMA,
        ],
        compiler_params=pltpu.CompilerParams(
            use_tc_tiling_on_sc=True, needs_layout_passes=False
        ),
    )
    wt = weights.T
    n_full = n_rows // _TB
    rem = n_rows - n_full * _TB
    tail = lax.slice(wt, (0, n_full * _TB), (dim, n_rows))
    tail = jnp.pad(tail, ((0, 0), (0, _TB - rem)))
    wrow = transpose(wt, tail).reshape(n_rows, dim)

    lookup = pl.kernel(
        functools.partial(_emb_body, n_groups, fields, dim),
        out_type=jax.ShapeDtypeStruct((batch, fields, dim), jnp.float32),
        mesh=mesh,
        scratch_types=[
            pltpu.VMEM(((total // _NW) // _CHUNK, _CHUNK), jnp.int32),
            pltpu.VMEM((_GROUP, dim), jnp.float32),
            pltpu.VMEM((_GROUP, dim), jnp.float32),
            pltpu.SemaphoreType.DMA,
            pltpu.SemaphoreType.DMA,
            pltpu.SemaphoreType.DMA,
            pltpu.SemaphoreType.DMA,
        ],
        compiler_params=pltpu.CompilerParams(use_tc_tiling_on_sc=False),
    )
    return lookup(wrow, idx3d)


def kernel(weights, token_ids):
    batch, fields = token_ids.shape
    dim = weights.shape[1]
    total = batch * fields
    per_w = total // _NW
    n_groups = per_w // _GROUP
    flat = token_ids.reshape(total)
    return _emb_lookup(weights, flat, n_groups, fields, dim)


# speedup vs baseline: 2.3674x; 1.4898x over previous
import functools

import jax
import jax.numpy as jnp
from jax import lax
from jax.experimental import pallas as pl
from jax.experimental.pallas import tpu as pltpu
from jax.experimental.pallas import tpu_sc as plsc

_NC = 2
_NS = 16
_NW = _NC * _NS

_CHUNK = 104
_SUPER = 4
_GROUP = _CHUNK * _SUPER


def _emb_body(n_groups, fields, dim, table_hbm, idx_hbm, out_hbm,
              idx_v, buf0, buf1, gs0, gs1, os0, os1):
    wid = lax.axis_index("s") * _NC + lax.axis_index("c")
    per_w = n_groups * _GROUP
    b_per_g = _GROUP // fields
    base_b = wid * (per_w // fields)
    pltpu.sync_copy(idx_hbm.at[wid], idx_v)

    bufs = (buf0, buf1)
    gsems = (gs0, gs1)
    osems = (os0, os1)

    def fire(g):
        b = g % 2
        return [
            pltpu.async_copy(
                table_hbm.at[idx_v.at[g * _SUPER + k]],
                bufs[b].at[pl.ds(k * _CHUNK, _CHUNK)],
                gsems[b],
            )
            for k in range(_SUPER)
        ]

    def drain_out(g):
        b = g % 2
        return [
            pltpu.async_copy(
                bufs[b].at[pl.ds(j * fields, fields)],
                out_hbm.at[base_b + g * b_per_g + j],
                osems[b],
            )
            for j in range(b_per_g)
        ]

    pend = {0: fire(0), 1: fire(1)}
    tail = []
    for g in range(n_groups):
        b = g % 2
        for cp in pend[b]:
            cp.wait()
        ocps = drain_out(g)
        if g + 2 < n_groups:
            for ocp in ocps:
                ocp.wait()
            pend[b] = fire(g + 2)
        else:
            tail.extend(ocps)
    for ocp in tail:
        ocp.wait()


@functools.partial(jax.jit, static_argnums=(2, 3, 4))
def _emb_lookup(weights, flat_idx, n_groups, fields, dim):
    total = flat_idx.shape[0]
    batch = total // fields
    idx3d = flat_idx.reshape(_NW, (total // _NW) // _CHUNK, _CHUNK)
    mesh = plsc.VectorSubcoreMesh(core_axis_name="c", subcore_axis_name="s")
    run = pl.kernel(
        functools.partial(_emb_body, n_groups, fields, dim),
        out_type=jax.ShapeDtypeStruct((batch, fields, dim), jnp.float32),
        mesh=mesh,
        scratch_types=[
            pltpu.VMEM(((total // _NW) // _CHUNK, _CHUNK), jnp.int32),
            pltpu.VMEM((_GROUP, dim), jnp.float32),
            pltpu.VMEM((_GROUP, dim), jnp.float32),
            pltpu.SemaphoreType.DMA,
            pltpu.SemaphoreType.DMA,
            pltpu.SemaphoreType.DMA,
            pltpu.SemaphoreType.DMA,
        ],
        compiler_params=pltpu.CompilerParams(use_tc_tiling_on_sc=False),
    )
    return run(weights, idx3d)


def kernel(weights, token_ids):
    batch, fields = token_ids.shape
    dim = weights.shape[1]
    total = batch * fields
    per_w = total // _NW
    n_groups = per_w // _GROUP
    flat = token_ids.reshape(total)
    return _emb_lookup(weights, flat, n_groups, fields, dim)
